# Initial kernel scaffold; baseline (speedup 1.0000x reference)
import jax
import jax.numpy as jnp
from jax.experimental import pallas as pl


def kernel(user_feat, item_feat, weight, bias):
    raise NotImplementedError("write your pallas kernel here")


# speedup vs baseline: 1.5309x; 1.5309x over previous
import functools

import jax
import jax.numpy as jnp
from jax import lax
from jax.experimental import pallas as pl
from jax.experimental.pallas import tpu as pltpu


def _round_up(x: int, m: int) -> int:
    return ((x + m - 1) // m) * m


def _linear_kernel(u_ref, w_ref, b_ref, uf_ref, ufb_ref):
    u = u_ref[...].astype(jnp.bfloat16)
    w = w_ref[...].astype(jnp.bfloat16)
    uf = lax.dot_general(
        u, w,
        dimension_numbers=(((1,), (1,)), ((), ())),
        preferred_element_type=jnp.float32,
    ) + b_ref[...]
    uf_ref[...] = uf.astype(uf_ref.dtype)
    ufb_ref[...] = uf.astype(jnp.bfloat16)


def _score_kernel(tu: int, ufb_ref, item_ref, res_ref):
    i = pl.program_id(1)
    u = ufb_ref[pl.ds(i * tu, tu), :]
    it = item_ref[...].astype(jnp.bfloat16)
    res = lax.dot_general(
        u, it,
        dimension_numbers=(((1,), (1,)), ((), ())),
        preferred_element_type=jnp.float32,
    )
    res_ref[...] = res.astype(res_ref.dtype)


@jax.jit
def _forward(user_feat, item_feat, weight, bias):
    U, F = user_feat.shape
    I, _ = item_feat.shape
    isz = jnp.dtype(user_feat.dtype).itemsize

    tu = min(1024, _round_up(U, 8))
    ti = min(2048, _round_up(I, 128))
    U_pad = _round_up(U, tu)
    I_pad = _round_up(I, ti)

    user_p = user_feat if U_pad == U else jnp.pad(user_feat, ((0, U_pad - U), (0, 0)))
    item_p = item_feat if I_pad == I else jnp.pad(item_feat, ((0, I_pad - I), (0, 0)))
    bias2d = bias.reshape(1, F)

    ufeat_p, ufeat_b = pl.pallas_call(
        _linear_kernel,
        out_shape=(
            jax.ShapeDtypeStruct((U_pad, F), user_feat.dtype),
            jax.ShapeDtypeStruct((U_pad, F), jnp.bfloat16),
        ),
        grid=(U_pad // tu,),
        in_specs=[
            pl.BlockSpec((tu, F), lambda i: (i, 0)),
            pl.BlockSpec((F, F), lambda i: (0, 0)),
            pl.BlockSpec((1, F), lambda i: (0, 0)),
        ],
        out_specs=(
            pl.BlockSpec((tu, F), lambda i: (i, 0)),
            pl.BlockSpec((tu, F), lambda i: (i, 0)),
        ),
        compiler_params=pltpu.CompilerParams(
            dimension_semantics=("parallel",),
            vmem_limit_bytes=64 * 1024 * 1024,
        ),
        cost_estimate=pl.CostEstimate(
            flops=2 * U_pad * F * F,
            transcendentals=0,
            bytes_accessed=isz * (2 * U_pad * F + F * F + F) + 2 * U_pad * F,
        ),
    )(user_p, weight, bias2d)

    res_p = pl.pallas_call(
        functools.partial(_score_kernel, tu),
        out_shape=jax.ShapeDtypeStruct((U_pad, I_pad), user_feat.dtype),
        grid=(I_pad // ti, U_pad // tu),
        in_specs=[
            pl.BlockSpec((U_pad, F), lambda j, i: (0, 0)),
            pl.BlockSpec((ti, F), lambda j, i: (j, 0)),
        ],
        out_specs=pl.BlockSpec((tu, ti), lambda j, i: (i, j)),
        compiler_params=pltpu.CompilerParams(
            dimension_semantics=("parallel", "arbitrary"),
            vmem_limit_bytes=64 * 1024 * 1024,
        ),
        cost_estimate=pl.CostEstimate(
            flops=2 * U_pad * I_pad * F,
            transcendentals=0,
            bytes_accessed=isz * (I_pad * F + U_pad * I_pad) + U_pad * F,
        ),
    )(ufeat_b, item_p)

    ufeat = ufeat_p if U_pad == U else ufeat_p[:U, :]
    result = res_p if (U_pad == U and I_pad == I) else res_p[:U, :I]
    return ufeat, result


def kernel(user_feat, item_feat, weight, bias):
    return _forward(user_feat, item_feat, weight, bias)


# speedup vs baseline: 1.8091x; 1.1817x over previous
import jax
import jax.numpy as jnp
from jax import lax
from jax.experimental import pallas as pl
from jax.experimental.pallas import tpu as pltpu


def _round_up(x: int, m: int) -> int:
    return ((x + m - 1) // m) * m


def _fused_kernel(u_ref, w_ref, b_ref, item_ref, uf_ref, res_ref):
    uf = lax.dot_general(
        u_ref[...], w_ref[...],
        dimension_numbers=(((1,), (1,)), ((), ())),
        preferred_element_type=jnp.float32,
    ) + b_ref[...]
    uf_ref[...] = uf.astype(uf_ref.dtype)
    res = lax.dot_general(
        uf, item_ref[...],
        dimension_numbers=(((1,), (1,)), ((), ())),
        preferred_element_type=jnp.float32,
    )
    res_ref[...] = res.astype(res_ref.dtype)


@jax.jit
def _forward(user_feat, item_feat, weight, bias):
    U, F = user_feat.shape
    I, _ = item_feat.shape
    isz = jnp.dtype(user_feat.dtype).itemsize

    tu = min(512, _round_up(U, 8))
    U_pad = _round_up(U, tu)

    user_p = user_feat if U_pad == U else jnp.pad(user_feat, ((0, U_pad - U), (0, 0)))
    bias2d = bias.reshape(1, F)

    ufeat_p, result = pl.pallas_call(
        _fused_kernel,
        out_shape=(
            jax.ShapeDtypeStruct((U_pad, F), user_feat.dtype),
            jax.ShapeDtypeStruct((U_pad, I), user_feat.dtype),
        ),
        grid=(U_pad // tu,),
        in_specs=[
            pl.BlockSpec((tu, F), lambda i: (i, 0)),
            pl.BlockSpec((F, F), lambda i: (0, 0)),
            pl.BlockSpec((1, F), lambda i: (0, 0)),
            pl.BlockSpec((I, F), lambda i: (0, 0)),
        ],
        out_specs=(
            pl.BlockSpec((tu, F), lambda i: (i, 0)),
            pl.BlockSpec((tu, I), lambda i: (i, 0)),
        ),
        compiler_params=pltpu.CompilerParams(
            dimension_semantics=("parallel",),
            vmem_limit_bytes=110 * 1024 * 1024,
        ),
        cost_estimate=pl.CostEstimate(
            flops=2 * U_pad * F * (F + I),
            transcendentals=0,
            bytes_accessed=isz * (U_pad * F * 2 + F * F + F + I * F + U_pad * I),
        ),
    )(user_p, weight, bias2d, item_feat)

    ufeat = ufeat_p if U_pad == U else ufeat_p[:U, :]
    result = result if U_pad == U else result[:U, :]
    return ufeat, result


def kernel(user_feat, item_feat, weight, bias):
    return _forward(user_feat, item_feat, weight, bias)
